# baseline (speedup 1.0000x reference)
import jax
import jax.numpy as jnp
from jax.experimental import pallas as pl
from jax.experimental.pallas import tpu as pltpu

RADIUS = 0.1
BLOCK_R = 256


def _loss_kernel(s_ref, ss_ref, dT_ref, ds_ref, out_ref, num_acc, cnt_acc):
    i = pl.program_id(0)
    nsteps = pl.num_programs(0)

    s = s_ref[...]
    dT = dT_ref[...]
    dot2 = jnp.dot(s, dT, preferred_element_type=jnp.float32)
    sq_s = jnp.sum(s * s, axis=1, keepdims=True)
    sq_d = jnp.sum(dT * dT, axis=0, keepdims=True)
    d2 = sq_s + sq_d - 2.0 * dot2

    mask = d2 < (RADIUS * RADIUS)
    diff = ss_ref[...] - ds_ref[...]
    contrib = jnp.where(mask, diff * diff, 0.0)
    maskf = jnp.where(mask, 1.0, 0.0)

    num = jnp.sum(contrib)
    cnt = jnp.sum(maskf)

    @pl.when(i == 0)
    def _init():
        num_acc[0, 0] = num
        cnt_acc[0, 0] = cnt

    @pl.when(i != 0)
    def _accum():
        num_acc[0, 0] += num
        cnt_acc[0, 0] += cnt

    @pl.when(i == nsteps - 1)
    def _finish():
        loss = num_acc[0, 0] / jnp.maximum(cnt_acc[0, 0], 1.0)
        out_ref[...] = jnp.full((1, 1), loss, dtype=jnp.float32)


def kernel(src_xyz, src_scores, dst_xyz, dst_scores):
    n = src_xyz.shape[0]
    m = dst_xyz.shape[0]
    ss = src_scores.reshape(n, 1)
    ds = dst_scores.reshape(1, m)
    dT = dst_xyz.T

    grid = (n // BLOCK_R,)
    out = pl.pallas_call(
        _loss_kernel,
        grid=grid,
        in_specs=[
            pl.BlockSpec((BLOCK_R, 3), lambda i: (i, 0)),
            pl.BlockSpec((BLOCK_R, 1), lambda i: (i, 0)),
            pl.BlockSpec((3, m), lambda i: (0, 0)),
            pl.BlockSpec((1, m), lambda i: (0, 0)),
        ],
        out_specs=pl.BlockSpec((1, 1), lambda i: (0, 0)),
        out_shape=jax.ShapeDtypeStruct((1, 1), jnp.float32),
        scratch_shapes=[
            pltpu.SMEM((1, 1), jnp.float32),
            pltpu.SMEM((1, 1), jnp.float32),
        ],
    )(src_xyz, ss, dT, ds)
    return out[0, 0]
